# baseline (speedup 1.0000x reference)
import functools

import jax
import jax.numpy as jnp
from jax import lax
from jax.experimental import pallas as pl
from jax.experimental.pallas import tpu as pltpu
from jax.experimental.pallas import tpu_sc as plsc

K_HOPS = 10
ALPHA = 0.05
NCORES = 2
NSUB = 16
CH = 128
IC = 8
NBUF = 4
RB = 40
RPT = 16


def _f32v(val):
    return jnp.full((16,), val, dtype=jnp.float32)


def _ssgc_sc(x0, x1, row2d, col2d, *, n, d, cpb):
    fpc = d // NCORES
    c1 = (1.0 - ALPHA) / K_HOPS
    nch_real = n // RB
    npad = -(-(n + 1) // RB) * RB
    nzch = npad // RB
    ndeg = NSUB * RB * RPT
    ngroups = cpb // IC
    npair = ngroups // 2

    mesh = plsc.VectorSubcoreMesh(
        core_axis_name="c", subcore_axis_name="s", num_cores=NCORES,
        num_subcores=NSUB)

    @functools.partial(
        pl.kernel,
        out_type=[jax.ShapeDtypeStruct((n, fpc), jnp.float32),
                  jax.ShapeDtypeStruct((n, fpc), jnp.float32)],
        mesh=mesh,
        compiler_params=pltpu.CompilerParams(use_tc_tiling_on_sc=False),
        scratch_types=[
            pltpu.VMEM_SHARED((npad, fpc), jnp.float32),
            pltpu.VMEM_SHARED((npad, fpc), jnp.float32),
            pltpu.VMEM_SHARED((ndeg,), jnp.float32),
            pltpu.VMEM((IC, CH), jnp.int32),
            pltpu.VMEM((IC, CH), jnp.int32),
            pltpu.VMEM((IC, CH), jnp.int32),
            pltpu.VMEM((IC, CH), jnp.int32),
            [pltpu.VMEM((CH, fpc), jnp.float32) for _ in range(NBUF)],
            pltpu.VMEM((RB, fpc), jnp.float32),
            pltpu.VMEM((RB, fpc), jnp.float32),
            pltpu.VMEM((RB, fpc), jnp.float32),
            pltpu.VMEM((CH,), jnp.float32),
            pltpu.VMEM((RB * RPT,), jnp.float32),
            pltpu.VMEM((RB * RPT + 16,), jnp.float32),
            pltpu.VMEM((RB * RPT + 16,), jnp.float32),
            pltpu.VMEM((RB * RPT + 16,), jnp.float32),
            pltpu.VMEM((RB * RPT,), jnp.float32),
            [pltpu.SemaphoreType.DMA for _ in range(NBUF)],
            [pltpu.SemaphoreType.DMA for _ in range(NBUF)],
            [pltpu.SemaphoreType.DMA for _ in range(4)],
            [pltpu.SemaphoreType.DMA for _ in range(3)],
        ],
    )
    def k(x0_hbm, x1_hbm, row_hbm, col_hbm, h0_hbm, h1_hbm,
          z_sh, u_sh, deg_sh, rowca, colca, rowcb, colcb, gb,
          ubuf, zbuf, szbuf, onesb,
          degb, dinvb, d2b, dsqb, zdegb, gsem, ssem, isem, msem):
        cid = lax.axis_index("c")
        sid = lax.axis_index("s")
        r0 = sid * (RB * RPT)
        g0 = sid * cpb
        nreal = jnp.minimum(RPT, jnp.maximum(0, nch_real - sid * RPT))
        nzero = jnp.minimum(RPT, jnp.maximum(0, nzch - sid * RPT))

        def per_core(fn0, fn1):
            def w0():
                fn0()
            def w1():
                fn1()
            pl.when(cid == 0)(w0)
            pl.when(cid == 1)(w1)

        def fill_zero_ubuf(i, carry):
            def fill_f(f, carry2):
                ubuf[i, pl.ds(f * 16, 16)] = _f32v(0.0)
                return carry2
            return lax.fori_loop(0, fpc // 16, fill_f, carry, unroll=False)
        lax.fori_loop(0, RB, fill_zero_ubuf, 0, unroll=False)

        def fill_ones(i, carry):
            onesb[pl.ds(i * 16, 16)] = _f32v(1.0)
            return carry
        lax.fori_loop(0, CH // 16, fill_ones, 0, unroll=False)

        def fill_zdeg(i, carry):
            zdegb[pl.ds(i * 16, 16)] = _f32v(0.0)
            return carry
        lax.fori_loop(0, RB * RPT // 16, fill_zdeg, 0, unroll=False)

        pltpu.sync_copy(zdegb, deg_sh.at[pl.ds(r0, RB * RPT)])

        def zero_tables(c, carry):
            rr = r0 + c * RB
            pltpu.sync_copy(ubuf, u_sh.at[pl.ds(rr, RB), :])
            pltpu.sync_copy(ubuf, z_sh.at[pl.ds(rr, RB), :])
            return carry
        lax.fori_loop(0, nzero, zero_tables, 0, unroll=False)
        plsc.subcore_barrier()

        def deg_group(g, carry):
            pltpu.sync_copy(col_hbm.at[pl.ds(g0 + g * IC, IC), :], colca)
            def deg_chunk(j, carry2):
                pltpu.sync_copy(onesb, deg_sh.at[colca.at[j]], add=True)
                return carry2
            return lax.fori_loop(0, IC, deg_chunk, carry, unroll=False)
        lax.fori_loop(0, ngroups, deg_group, 0, unroll=False)
        plsc.subcore_barrier()

        pltpu.sync_copy(deg_sh.at[pl.ds(r0, RB * RPT)], degb)
        def rsqrt_vec(v, carry):
            dv = degb[pl.ds(v * 16, 16)] + _f32v(1.0)
            iv = lax.bitcast_convert_type(dv, jnp.int32)
            iv = jnp.int32(0x5F3759DF) - lax.shift_right_arithmetic(
                iv, jnp.int32(1))
            y = lax.bitcast_convert_type(iv, jnp.float32)
            half_d = dv * _f32v(0.5)
            y = y * (_f32v(1.5) - half_d * y * y)
            y = y * (_f32v(1.5) - half_d * y * y)
            y = y * (_f32v(1.5) - half_d * y * y)
            dinvb[pl.ds(v * 16, 16)] = y
            d2b[pl.ds(v * 16, 16)] = _f32v(1.0) / dv
            dsqb[pl.ds(v * 16, 16)] = dv * y * _f32v(c1)
            return carry
        lax.fori_loop(0, RB * RPT // 16, rsqrt_vec, 0, unroll=False)

        a_over_c = ALPHA / c1
        def z0_chunk(c, carry):
            rr = r0 + c * RB
            per_core(
                lambda: pltpu.sync_copy(x0_hbm.at[pl.ds(rr, RB), :], zbuf),
                lambda: pltpu.sync_copy(x1_hbm.at[pl.ds(rr, RB), :], zbuf))
            def z0_row(i, carry2):
                s = dinvb[pl.ds(c * RB + i, 16)][0]
                def z0_f(f, carry3):
                    sl = pl.ds(f * 16, 16)
                    zv = zbuf[i, sl] * s
                    zbuf[i, sl] = zv
                    szbuf[i, sl] = zv * a_over_c
                    return carry3
                return lax.fori_loop(0, fpc // 16, z0_f, carry2, unroll=False)
            lax.fori_loop(0, RB, z0_row, 0, unroll=False)
            pltpu.sync_copy(zbuf, z_sh.at[pl.ds(rr, RB), :])
            per_core(
                lambda: pltpu.sync_copy(szbuf, h0_hbm.at[pl.ds(rr, RB), :]),
                lambda: pltpu.sync_copy(szbuf, h1_hbm.at[pl.ds(rr, RB), :]))
            return carry
        lax.fori_loop(0, nreal, z0_chunk, 0, unroll=False)
        plsc.subcore_barrier()

        idxbufs = ((rowca, colca), (rowcb, colcb))

        def idx_fetch(g, par):
            rowc, colc = idxbufs[par]
            pltpu.async_copy(row_hbm.at[pl.ds(g0 + g * IC, IC), :], rowc,
                             isem[2 * par])
            pltpu.async_copy(col_hbm.at[pl.ds(g0 + g * IC, IC), :], colc,
                             isem[2 * par + 1])

        def idx_wait(g, par):
            rowc, colc = idxbufs[par]
            pltpu.make_async_copy(row_hbm.at[pl.ds(g0 + g * IC, IC), :],
                                  rowc, isem[2 * par]).wait()
            pltpu.make_async_copy(col_hbm.at[pl.ds(g0 + g * IC, IC), :],
                                  colc, isem[2 * par + 1]).wait()

        def gather_issue(rowc, lj, b):
            pltpu.async_copy(z_sh.at[rowc.at[lj]], gb[b], gsem[b])

        def gather_wait(rowc, lj, b):
            pltpu.make_async_copy(z_sh.at[rowc.at[lj]], gb[b],
                                  gsem[b]).wait()

        def scatter_issue(colc, lj, b):
            pltpu.async_copy(gb[b], u_sh.at[colc.at[lj]], ssem[b], add=True)

        def scatter_wait(colc, lj, b):
            pltpu.make_async_copy(gb[b], u_sh.at[colc.at[lj]],
                                  ssem[b]).wait()

        def process_group(g, par, first):
            rowc, colc = idxbufs[par]
            idx_wait(g, par)
            for lj in (0, 1):
                def _w(lj=lj):
                    scatter_wait(colc, lj, lj)
                if first:
                    pl.when(g >= 1)(_w)
                else:
                    _w()
                gather_issue(rowc, lj, lj)
            for lj in range(IC):
                b = lj % NBUF
                gather_wait(rowc, lj, b)
                scatter_issue(colc, lj, b)
                if lj <= IC - 3:
                    b2 = (lj + 2) % NBUF
                    if lj >= 2:
                        scatter_wait(colc, lj - 2, b2)
                    else:
                        def _w2(lj=lj, b2=b2):
                            scatter_wait(colc, lj, b2)
                        if first:
                            pl.when(g >= 1)(_w2)
                        else:
                            _w2()
                    gather_issue(rowc, lj + 2, b2)

        def hop(_, carry):
            idx_fetch(0, 0)
            idx_fetch(1, 1)

            def pair(p, carry2):
                process_group(2 * p, 0, True)
                @pl.when(p < npair - 1)
                def _():
                    idx_fetch(2 * p + 2, 0)
                process_group(2 * p + 1, 1, False)
                @pl.when(p < npair - 1)
                def _():
                    idx_fetch(2 * p + 3, 1)
                return carry2
            lax.fori_loop(0, npair, pair, 0, unroll=False)

            for lj in range(IC - NBUF, IC):
                scatter_wait(colcb, lj, lj % NBUF)

            plsc.subcore_barrier()

            def scale_chunk(c, carry2):
                rr = r0 + c * RB
                du = pltpu.async_copy(u_sh.at[pl.ds(rr, RB), :], ubuf,
                                      msem[0])
                dz = pltpu.async_copy(z_sh.at[pl.ds(rr, RB), :], zbuf,
                                      msem[1])
                per_core(
                    lambda: pltpu.async_copy(h0_hbm.at[pl.ds(rr, RB), :],
                                             szbuf, msem[2]),
                    lambda: pltpu.async_copy(h1_hbm.at[pl.ds(rr, RB), :],
                                             szbuf, msem[2]))
                du.wait()
                dz.wait()
                pltpu.make_async_copy(h0_hbm.at[pl.ds(rr, RB), :], szbuf,
                                      msem[2]).wait()
                def scale_row(i, carry3):
                    s = d2b[pl.ds(c * RB + i, 16)][0]
                    def scale_f(f, carry4):
                        sl = pl.ds(f * 16, 16)
                        zv = (ubuf[i, sl] + zbuf[i, sl]) * s
                        zbuf[i, sl] = zv
                        szbuf[i, sl] = szbuf[i, sl] + zv
                        return carry4
                    return lax.fori_loop(0, fpc // 16, scale_f, carry3,
                                         unroll=False)
                lax.fori_loop(0, RB, scale_row, 0, unroll=False)
                lax.fori_loop(0, RB, fill_zero_ubuf, 0, unroll=False)
                du2 = pltpu.async_copy(ubuf, u_sh.at[pl.ds(rr, RB), :],
                                       msem[0])
                dz2 = pltpu.async_copy(zbuf, z_sh.at[pl.ds(rr, RB), :],
                                       msem[1])
                per_core(
                    lambda: pltpu.async_copy(szbuf,
                                             h0_hbm.at[pl.ds(rr, RB), :],
                                             msem[2]),
                    lambda: pltpu.async_copy(szbuf,
                                             h1_hbm.at[pl.ds(rr, RB), :],
                                             msem[2]))
                du2.wait()
                dz2.wait()
                pltpu.make_async_copy(szbuf, h0_hbm.at[pl.ds(rr, RB), :],
                                      msem[2]).wait()
                return carry2
            lax.fori_loop(0, nreal, scale_chunk, 0, unroll=False)
            plsc.subcore_barrier()
            return carry
        lax.fori_loop(0, K_HOPS, hop, 0, unroll=False)

        def out_chunk(c, carry):
            rr = r0 + c * RB
            per_core(
                lambda: pltpu.sync_copy(h0_hbm.at[pl.ds(rr, RB), :], szbuf),
                lambda: pltpu.sync_copy(h1_hbm.at[pl.ds(rr, RB), :], szbuf))
            def out_row(i, carry2):
                s = dsqb[pl.ds(c * RB + i, 16)][0]
                def out_f(f, carry3):
                    sl = pl.ds(f * 16, 16)
                    szbuf[i, sl] = szbuf[i, sl] * s
                    return carry3
                return lax.fori_loop(0, fpc // 16, out_f, carry2,
                                     unroll=False)
            lax.fori_loop(0, RB, out_row, 0, unroll=False)
            per_core(
                lambda: pltpu.sync_copy(szbuf, h0_hbm.at[pl.ds(rr, RB), :]),
                lambda: pltpu.sync_copy(szbuf, h1_hbm.at[pl.ds(rr, RB), :]))
            return carry
        lax.fori_loop(0, nreal, out_chunk, 0, unroll=False)

    return k(x0, x1, row2d, col2d)


def _matmul_tc(h0, h1, w, b2, *, n, d, c):
    bn = 400
    hd = d // 2

    def mm(h0_ref, h1_ref, w_ref, b_ref, o_ref):
        wv = w_ref[...]
        o_ref[...] = (
            jnp.dot(h0_ref[...], wv[:hd], preferred_element_type=jnp.float32)
            + jnp.dot(h1_ref[...], wv[hd:], preferred_element_type=jnp.float32)
            + b_ref[...])

    return pl.pallas_call(
        mm,
        grid=(n // bn,),
        in_specs=[
            pl.BlockSpec((bn, hd), lambda i: (i, 0)),
            pl.BlockSpec((bn, hd), lambda i: (i, 0)),
            pl.BlockSpec((d, c), lambda i: (0, 0)),
            pl.BlockSpec((1, c), lambda i: (0, 0)),
        ],
        out_specs=pl.BlockSpec((bn, c), lambda i: (i, 0)),
        out_shape=jax.ShapeDtypeStruct((n, c), jnp.float32),
    )(h0, h1, w, b2)


def kernel(x, edge_index, W, b):
    n, d = x.shape
    e = edge_index.shape[1]
    c = W.shape[1]

    cpb = -(-e // (NSUB * CH))
    cpb = -(-cpb // (2 * IC)) * (2 * IC)
    epad = cpb * NSUB * CH
    pad = jnp.full((epad - e,), n, dtype=jnp.int32)
    row2d = jnp.concatenate([edge_index[0], pad]).reshape(cpb * NSUB, CH)
    col2d = jnp.concatenate([edge_index[1], pad]).reshape(cpb * NSUB, CH)

    hd = d // 2
    h0, h1 = _ssgc_sc(x[:, :hd], x[:, hd:], row2d, col2d,
                      n=n, d=d, cpb=cpb)
    return _matmul_tc(h0, h1, W, b.reshape(1, c), n=n, d=d, c=c)


# speedup vs baseline: 1.0189x; 1.0189x over previous
import functools

import jax
import jax.numpy as jnp
from jax import lax
from jax.experimental import pallas as pl
from jax.experimental.pallas import tpu as pltpu
from jax.experimental.pallas import tpu_sc as plsc

K_HOPS = 10
ALPHA = 0.05
NCORES = 2
NSUB = 16
CH = 128
IC = 8
NBUF = 2
RB = 40
RPT = 16


def _f32v(val):
    return jnp.full((16,), val, dtype=jnp.float32)


def _ssgc_sc(x0, x1, row2d, col2d, *, n, d, cpb):
    fpc = d // NCORES
    c1 = (1.0 - ALPHA) / K_HOPS
    nch_real = n // RB
    npad = -(-(n + 1) // RB) * RB
    nzch = npad // RB
    ndeg = NSUB * RB * RPT
    ngroups = cpb // IC
    npair = ngroups // 2

    mesh = plsc.VectorSubcoreMesh(
        core_axis_name="c", subcore_axis_name="s", num_cores=NCORES,
        num_subcores=NSUB)

    @functools.partial(
        pl.kernel,
        out_type=[jax.ShapeDtypeStruct((n, fpc), jnp.float32),
                  jax.ShapeDtypeStruct((n, fpc), jnp.float32)],
        mesh=mesh,
        compiler_params=pltpu.CompilerParams(use_tc_tiling_on_sc=False),
        scratch_types=[
            pltpu.VMEM_SHARED((npad, fpc), jnp.float32),
            pltpu.VMEM_SHARED((npad, fpc), jnp.float32),
            pltpu.VMEM_SHARED((ndeg,), jnp.float32),
            pltpu.VMEM((IC, CH), jnp.int32),
            pltpu.VMEM((IC, CH), jnp.int32),
            pltpu.VMEM((IC, CH), jnp.int32),
            pltpu.VMEM((IC, CH), jnp.int32),
            [pltpu.VMEM((CH, fpc), jnp.float32) for _ in range(NBUF)],
            pltpu.VMEM((RB, fpc), jnp.float32),
            pltpu.VMEM((RB, fpc), jnp.float32),
            pltpu.VMEM((RB, fpc), jnp.float32),
            pltpu.VMEM((CH,), jnp.float32),
            pltpu.VMEM((RB * RPT,), jnp.float32),
            pltpu.VMEM((RB * RPT + 16,), jnp.float32),
            pltpu.VMEM((RB * RPT + 16,), jnp.float32),
            pltpu.VMEM((RB * RPT + 16,), jnp.float32),
            pltpu.VMEM((RB * RPT,), jnp.float32),
            [pltpu.SemaphoreType.DMA for _ in range(NBUF)],
            [pltpu.SemaphoreType.DMA for _ in range(NBUF)],
            [pltpu.SemaphoreType.DMA for _ in range(4)],
            [pltpu.SemaphoreType.DMA for _ in range(3)],
        ],
    )
    def k(x0_hbm, x1_hbm, row_hbm, col_hbm, h0_hbm, h1_hbm,
          z_sh, u_sh, deg_sh, rowca, colca, rowcb, colcb, gb,
          ubuf, zbuf, szbuf, onesb,
          degb, dinvb, d2b, dsqb, zdegb, gsem, ssem, isem, msem):
        cid = lax.axis_index("c")
        sid = lax.axis_index("s")
        r0 = sid * (RB * RPT)
        g0 = sid * cpb
        nreal = jnp.minimum(RPT, jnp.maximum(0, nch_real - sid * RPT))
        nzero = jnp.minimum(RPT, jnp.maximum(0, nzch - sid * RPT))

        def per_core(fn0, fn1):
            def w0():
                fn0()
            def w1():
                fn1()
            pl.when(cid == 0)(w0)
            pl.when(cid == 1)(w1)

        def fill_zero_ubuf(i, carry):
            def fill_f(f, carry2):
                ubuf[i, pl.ds(f * 16, 16)] = _f32v(0.0)
                return carry2
            return lax.fori_loop(0, fpc // 16, fill_f, carry, unroll=False)
        lax.fori_loop(0, RB, fill_zero_ubuf, 0, unroll=False)

        def fill_ones(i, carry):
            onesb[pl.ds(i * 16, 16)] = _f32v(1.0)
            return carry
        lax.fori_loop(0, CH // 16, fill_ones, 0, unroll=False)

        def fill_zdeg(i, carry):
            zdegb[pl.ds(i * 16, 16)] = _f32v(0.0)
            return carry
        lax.fori_loop(0, RB * RPT // 16, fill_zdeg, 0, unroll=False)

        pltpu.sync_copy(zdegb, deg_sh.at[pl.ds(r0, RB * RPT)])

        def zero_tables(c, carry):
            rr = r0 + c * RB
            pltpu.sync_copy(ubuf, u_sh.at[pl.ds(rr, RB), :])
            pltpu.sync_copy(ubuf, z_sh.at[pl.ds(rr, RB), :])
            return carry
        lax.fori_loop(0, nzero, zero_tables, 0, unroll=False)
        plsc.subcore_barrier()

        def deg_group(g, carry):
            pltpu.sync_copy(col_hbm.at[pl.ds(g0 + g * IC, IC), :], colca)
            def deg_chunk(j, carry2):
                pltpu.sync_copy(onesb, deg_sh.at[colca.at[j]], add=True)
                return carry2
            return lax.fori_loop(0, IC, deg_chunk, carry, unroll=False)
        lax.fori_loop(0, ngroups, deg_group, 0, unroll=False)
        plsc.subcore_barrier()

        pltpu.sync_copy(deg_sh.at[pl.ds(r0, RB * RPT)], degb)
        def rsqrt_vec(v, carry):
            dv = degb[pl.ds(v * 16, 16)] + _f32v(1.0)
            iv = lax.bitcast_convert_type(dv, jnp.int32)
            iv = jnp.int32(0x5F3759DF) - lax.shift_right_arithmetic(
                iv, jnp.int32(1))
            y = lax.bitcast_convert_type(iv, jnp.float32)
            half_d = dv * _f32v(0.5)
            y = y * (_f32v(1.5) - half_d * y * y)
            y = y * (_f32v(1.5) - half_d * y * y)
            y = y * (_f32v(1.5) - half_d * y * y)
            dinvb[pl.ds(v * 16, 16)] = y
            d2b[pl.ds(v * 16, 16)] = _f32v(1.0) / dv
            dsqb[pl.ds(v * 16, 16)] = dv * y * _f32v(c1)
            return carry
        lax.fori_loop(0, RB * RPT // 16, rsqrt_vec, 0, unroll=False)

        a_over_c = ALPHA / c1
        def z0_chunk(c, carry):
            rr = r0 + c * RB
            per_core(
                lambda: pltpu.sync_copy(x0_hbm.at[pl.ds(rr, RB), :], zbuf),
                lambda: pltpu.sync_copy(x1_hbm.at[pl.ds(rr, RB), :], zbuf))
            def z0_row(i, carry2):
                s = dinvb[pl.ds(c * RB + i, 16)][0]
                def z0_f(f, carry3):
                    sl = pl.ds(f * 16, 16)
                    zv = zbuf[i, sl] * s
                    zbuf[i, sl] = zv
                    szbuf[i, sl] = zv * a_over_c
                    return carry3
                return lax.fori_loop(0, fpc // 16, z0_f, carry2, unroll=False)
            lax.fori_loop(0, RB, z0_row, 0, unroll=False)
            pltpu.sync_copy(zbuf, z_sh.at[pl.ds(rr, RB), :])
            per_core(
                lambda: pltpu.sync_copy(szbuf, h0_hbm.at[pl.ds(rr, RB), :]),
                lambda: pltpu.sync_copy(szbuf, h1_hbm.at[pl.ds(rr, RB), :]))
            return carry
        lax.fori_loop(0, nreal, z0_chunk, 0, unroll=False)
        plsc.subcore_barrier()

        idxbufs = ((rowca, colca), (rowcb, colcb))

        def idx_fetch(g, par):
            rowc, colc = idxbufs[par]
            pltpu.async_copy(row_hbm.at[pl.ds(g0 + g * IC, IC), :], rowc,
                             isem[2 * par])
            pltpu.async_copy(col_hbm.at[pl.ds(g0 + g * IC, IC), :], colc,
                             isem[2 * par + 1])

        def idx_wait(g, par):
            rowc, colc = idxbufs[par]
            pltpu.make_async_copy(row_hbm.at[pl.ds(g0 + g * IC, IC), :],
                                  rowc, isem[2 * par]).wait()
            pltpu.make_async_copy(col_hbm.at[pl.ds(g0 + g * IC, IC), :],
                                  colc, isem[2 * par + 1]).wait()

        def gather_issue(rowc, lj, b):
            pltpu.async_copy(z_sh.at[rowc.at[lj]], gb[b], gsem[b])

        def gather_wait(rowc, lj, b):
            pltpu.make_async_copy(z_sh.at[rowc.at[lj]], gb[b],
                                  gsem[b]).wait()

        def scatter_issue(colc, lj, b):
            pltpu.async_copy(gb[b], u_sh.at[colc.at[lj]], ssem[b], add=True)

        def scatter_wait(colc, lj, b):
            pltpu.make_async_copy(gb[b], u_sh.at[colc.at[lj]],
                                  ssem[b]).wait()

        def process_group(g, par, first):
            rowc, colc = idxbufs[par]
            idx_wait(g, par)
            def _w0():
                scatter_wait(colc, 0, 0)
            if first:
                pl.when(g >= 1)(_w0)
            else:
                _w0()
            gather_issue(rowc, 0, 0)
            for lj in range(IC):
                b = lj % NBUF
                gather_wait(rowc, lj, b)
                scatter_issue(colc, lj, b)
                if lj < IC - 1:
                    b2 = (lj + 1) % NBUF
                    if lj >= 1:
                        scatter_wait(colc, lj - 1, b2)
                    else:
                        def _w1():
                            scatter_wait(colc, 0, b2)
                        if first:
                            pl.when(g >= 1)(_w1)
                        else:
                            _w1()
                    gather_issue(rowc, lj + 1, b2)

        def hop(_, carry):
            idx_fetch(0, 0)
            idx_fetch(1, 1)

            def pair(p, carry2):
                process_group(2 * p, 0, True)
                @pl.when(p < npair - 1)
                def _():
                    idx_fetch(2 * p + 2, 0)
                process_group(2 * p + 1, 1, False)
                @pl.when(p < npair - 1)
                def _():
                    idx_fetch(2 * p + 3, 1)
                return carry2
            lax.fori_loop(0, npair, pair, 0, unroll=False)

            for lj in range(IC - NBUF, IC):
                scatter_wait(colcb, lj, lj % NBUF)

            plsc.subcore_barrier()

            def scale_chunk(c, carry2):
                rr = r0 + c * RB
                du = pltpu.async_copy(u_sh.at[pl.ds(rr, RB), :], ubuf,
                                      msem[0])
                dz = pltpu.async_copy(z_sh.at[pl.ds(rr, RB), :], zbuf,
                                      msem[1])
                per_core(
                    lambda: pltpu.async_copy(h0_hbm.at[pl.ds(rr, RB), :],
                                             szbuf, msem[2]),
                    lambda: pltpu.async_copy(h1_hbm.at[pl.ds(rr, RB), :],
                                             szbuf, msem[2]))
                du.wait()
                dz.wait()
                pltpu.make_async_copy(h0_hbm.at[pl.ds(rr, RB), :], szbuf,
                                      msem[2]).wait()
                def scale_row(i, carry3):
                    s = d2b[pl.ds(c * RB + i, 16)][0]
                    def scale_f(f, carry4):
                        sl = pl.ds(f * 16, 16)
                        zv = (ubuf[i, sl] + zbuf[i, sl]) * s
                        zbuf[i, sl] = zv
                        szbuf[i, sl] = szbuf[i, sl] + zv
                        return carry4
                    return lax.fori_loop(0, fpc // 16, scale_f, carry3,
                                         unroll=False)
                lax.fori_loop(0, RB, scale_row, 0, unroll=False)
                lax.fori_loop(0, RB, fill_zero_ubuf, 0, unroll=False)
                du2 = pltpu.async_copy(ubuf, u_sh.at[pl.ds(rr, RB), :],
                                       msem[0])
                dz2 = pltpu.async_copy(zbuf, z_sh.at[pl.ds(rr, RB), :],
                                       msem[1])
                per_core(
                    lambda: pltpu.async_copy(szbuf,
                                             h0_hbm.at[pl.ds(rr, RB), :],
                                             msem[2]),
                    lambda: pltpu.async_copy(szbuf,
                                             h1_hbm.at[pl.ds(rr, RB), :],
                                             msem[2]))
                du2.wait()
                dz2.wait()
                pltpu.make_async_copy(szbuf, h0_hbm.at[pl.ds(rr, RB), :],
                                      msem[2]).wait()
                return carry2
            lax.fori_loop(0, nreal, scale_chunk, 0, unroll=False)
            plsc.subcore_barrier()
            return carry
        lax.fori_loop(0, K_HOPS, hop, 0, unroll=False)

        def out_chunk(c, carry):
            rr = r0 + c * RB
            per_core(
                lambda: pltpu.sync_copy(h0_hbm.at[pl.ds(rr, RB), :], szbuf),
                lambda: pltpu.sync_copy(h1_hbm.at[pl.ds(rr, RB), :], szbuf))
            def out_row(i, carry2):
                s = dsqb[pl.ds(c * RB + i, 16)][0]
                def out_f(f, carry3):
                    sl = pl.ds(f * 16, 16)
                    szbuf[i, sl] = szbuf[i, sl] * s
                    return carry3
                return lax.fori_loop(0, fpc // 16, out_f, carry2,
                                     unroll=False)
            lax.fori_loop(0, RB, out_row, 0, unroll=False)
            per_core(
                lambda: pltpu.sync_copy(szbuf, h0_hbm.at[pl.ds(rr, RB), :]),
                lambda: pltpu.sync_copy(szbuf, h1_hbm.at[pl.ds(rr, RB), :]))
            return carry
        lax.fori_loop(0, nreal, out_chunk, 0, unroll=False)

    return k(x0, x1, row2d, col2d)


def _matmul_tc(h0, h1, w, b2, *, n, d, c):
    bn = 400
    hd = d // 2

    def mm(h0_ref, h1_ref, w_ref, b_ref, o_ref):
        wv = w_ref[...]
        o_ref[...] = (
            jnp.dot(h0_ref[...], wv[:hd], preferred_element_type=jnp.float32)
            + jnp.dot(h1_ref[...], wv[hd:], preferred_element_type=jnp.float32)
            + b_ref[...])

    return pl.pallas_call(
        mm,
        grid=(n // bn,),
        in_specs=[
            pl.BlockSpec((bn, hd), lambda i: (i, 0)),
            pl.BlockSpec((bn, hd), lambda i: (i, 0)),
            pl.BlockSpec((d, c), lambda i: (0, 0)),
            pl.BlockSpec((1, c), lambda i: (0, 0)),
        ],
        out_specs=pl.BlockSpec((bn, c), lambda i: (i, 0)),
        out_shape=jax.ShapeDtypeStruct((n, c), jnp.float32),
    )(h0, h1, w, b2)


def kernel(x, edge_index, W, b):
    n, d = x.shape
    e = edge_index.shape[1]
    c = W.shape[1]

    cpb = -(-e // (NSUB * CH))
    cpb = -(-cpb // (2 * IC)) * (2 * IC)
    epad = cpb * NSUB * CH
    pad = jnp.full((epad - e,), n, dtype=jnp.int32)
    row2d = jnp.concatenate([edge_index[0], pad]).reshape(cpb * NSUB, CH)
    col2d = jnp.concatenate([edge_index[1], pad]).reshape(cpb * NSUB, CH)

    hd = d // 2
    h0, h1 = _ssgc_sc(x[:, :hd], x[:, hd:], row2d, col2d,
                      n=n, d=d, cpb=cpb)
    return _matmul_tc(h0, h1, W, b.reshape(1, c), n=n, d=d, c=c)


# speedup vs baseline: 1.0753x; 1.0554x over previous
import functools

import jax
import jax.numpy as jnp
from jax import lax
from jax.experimental import pallas as pl
from jax.experimental.pallas import tpu as pltpu
from jax.experimental.pallas import tpu_sc as plsc

K_HOPS = 10
ALPHA = 0.05
NCORES = 2
NSUB = 16
CH = 128
IC = 16
NBUF = 2
RB = 80
RPT = 8


def _f32v(val):
    return jnp.full((16,), val, dtype=jnp.float32)


def _ssgc_sc(x0, x1, row2d, col2d, *, n, d, cpb):
    fpc = d // NCORES
    c1 = (1.0 - ALPHA) / K_HOPS
    nch_real = n // RB
    npad = -(-(n + 1) // RB) * RB
    nzch = npad // RB
    ndeg = NSUB * RB * RPT
    ngroups = cpb // IC
    npair = ngroups // 2

    mesh = plsc.VectorSubcoreMesh(
        core_axis_name="c", subcore_axis_name="s", num_cores=NCORES,
        num_subcores=NSUB)

    @functools.partial(
        pl.kernel,
        out_type=[jax.ShapeDtypeStruct((n, fpc), jnp.float32),
                  jax.ShapeDtypeStruct((n, fpc), jnp.float32)],
        mesh=mesh,
        compiler_params=pltpu.CompilerParams(use_tc_tiling_on_sc=False),
        scratch_types=[
            pltpu.VMEM_SHARED((npad, fpc), jnp.float32),
            pltpu.VMEM_SHARED((npad, fpc), jnp.float32),
            pltpu.VMEM_SHARED((ndeg,), jnp.float32),
            pltpu.VMEM((IC, CH), jnp.int32),
            pltpu.VMEM((IC, CH), jnp.int32),
            pltpu.VMEM((IC, CH), jnp.int32),
            pltpu.VMEM((IC, CH), jnp.int32),
            [pltpu.VMEM((CH, fpc), jnp.float32) for _ in range(NBUF)],
            pltpu.VMEM((RB, fpc), jnp.float32),
            pltpu.VMEM((RB, fpc), jnp.float32),
            pltpu.VMEM((RB, fpc), jnp.float32),
            pltpu.VMEM((RB, fpc), jnp.float32),
            pltpu.VMEM((CH,), jnp.float32),
            pltpu.VMEM((RB * RPT,), jnp.float32),
            pltpu.VMEM((RB * RPT + 16,), jnp.float32),
            pltpu.VMEM((RB * RPT + 16,), jnp.float32),
            pltpu.VMEM((RB * RPT + 16,), jnp.float32),
            pltpu.VMEM((RB * RPT,), jnp.float32),
            [pltpu.SemaphoreType.DMA for _ in range(NBUF)],
            [pltpu.SemaphoreType.DMA for _ in range(NBUF)],
            [pltpu.SemaphoreType.DMA for _ in range(4)],
            [pltpu.SemaphoreType.DMA for _ in range(3)],
        ],
    )
    def k(x0_hbm, x1_hbm, row_hbm, col_hbm, h0_hbm, h1_hbm,
          z_sh, u_sh, deg_sh, rowca, colca, rowcb, colcb, gb,
          ubuf, zbuf, szbuf, zerob, onesb,
          degb, dinvb, d2b, dsqb, zdegb, gsem, ssem, isem, msem):
        cid = lax.axis_index("c")
        sid = lax.axis_index("s")
        r0 = sid * (RB * RPT)
        g0 = sid * cpb
        nreal = jnp.minimum(RPT, jnp.maximum(0, nch_real - sid * RPT))
        nzero = jnp.minimum(RPT, jnp.maximum(0, nzch - sid * RPT))

        def per_core(fn0, fn1):
            def w0():
                fn0()
            def w1():
                fn1()
            pl.when(cid == 0)(w0)
            pl.when(cid == 1)(w1)

        def fill_const(i, carry):
            def fill_f(f, carry2):
                zerob[i, pl.ds(f * 16, 16)] = _f32v(0.0)
                return carry2
            return lax.fori_loop(0, fpc // 16, fill_f, carry, unroll=False)
        lax.fori_loop(0, RB, fill_const, 0, unroll=False)

        def fill_ones(i, carry):
            onesb[pl.ds(i * 16, 16)] = _f32v(1.0)
            return carry
        lax.fori_loop(0, CH // 16, fill_ones, 0, unroll=False)

        def fill_zdeg(i, carry):
            zdegb[pl.ds(i * 16, 16)] = _f32v(0.0)
            return carry
        lax.fori_loop(0, RB * RPT // 16, fill_zdeg, 0, unroll=False)

        pltpu.sync_copy(zdegb, deg_sh.at[pl.ds(r0, RB * RPT)])

        def zero_tables(c, carry):
            rr = r0 + c * RB
            pltpu.sync_copy(zerob, u_sh.at[pl.ds(rr, RB), :])
            pltpu.sync_copy(zerob, z_sh.at[pl.ds(rr, RB), :])
            return carry
        lax.fori_loop(0, nzero, zero_tables, 0, unroll=False)
        plsc.subcore_barrier()

        def deg_group(g, carry):
            pltpu.sync_copy(col_hbm.at[pl.ds(g0 + g * IC, IC), :], colca)
            def deg_chunk(j, carry2):
                pltpu.sync_copy(onesb, deg_sh.at[colca.at[j]], add=True)
                return carry2
            return lax.fori_loop(0, IC, deg_chunk, carry, unroll=False)
        lax.fori_loop(0, ngroups, deg_group, 0, unroll=False)
        plsc.subcore_barrier()

        pltpu.sync_copy(deg_sh.at[pl.ds(r0, RB * RPT)], degb)
        def rsqrt_vec(v, carry):
            dv = degb[pl.ds(v * 16, 16)] + _f32v(1.0)
            iv = lax.bitcast_convert_type(dv, jnp.int32)
            iv = jnp.int32(0x5F3759DF) - lax.shift_right_arithmetic(
                iv, jnp.int32(1))
            y = lax.bitcast_convert_type(iv, jnp.float32)
            half_d = dv * _f32v(0.5)
            y = y * (_f32v(1.5) - half_d * y * y)
            y = y * (_f32v(1.5) - half_d * y * y)
            y = y * (_f32v(1.5) - half_d * y * y)
            dinvb[pl.ds(v * 16, 16)] = y
            d2b[pl.ds(v * 16, 16)] = _f32v(1.0) / dv
            dsqb[pl.ds(v * 16, 16)] = dv * y * _f32v(c1)
            return carry
        lax.fori_loop(0, RB * RPT // 16, rsqrt_vec, 0, unroll=False)

        a_over_c = ALPHA / c1
        def z0_chunk(c, carry):
            rr = r0 + c * RB
            per_core(
                lambda: pltpu.sync_copy(x0_hbm.at[pl.ds(rr, RB), :], zbuf),
                lambda: pltpu.sync_copy(x1_hbm.at[pl.ds(rr, RB), :], zbuf))
            def z0_row(i, carry2):
                s = dinvb[pl.ds(c * RB + i, 16)][0]
                def z0_f(f, carry3):
                    sl = pl.ds(f * 16, 16)
                    zv = zbuf[i, sl] * s
                    zbuf[i, sl] = zv
                    szbuf[i, sl] = zv * a_over_c
                    return carry3
                return lax.fori_loop(0, fpc // 16, z0_f, carry2, unroll=False)
            lax.fori_loop(0, RB, z0_row, 0, unroll=False)
            pltpu.sync_copy(zbuf, z_sh.at[pl.ds(rr, RB), :])
            per_core(
                lambda: pltpu.sync_copy(szbuf, h0_hbm.at[pl.ds(rr, RB), :]),
                lambda: pltpu.sync_copy(szbuf, h1_hbm.at[pl.ds(rr, RB), :]))
            return carry
        lax.fori_loop(0, nreal, z0_chunk, 0, unroll=False)
        plsc.subcore_barrier()

        idxbufs = ((rowca, colca), (rowcb, colcb))

        def idx_fetch(g, par):
            rowc, colc = idxbufs[par]
            pltpu.async_copy(row_hbm.at[pl.ds(g0 + g * IC, IC), :], rowc,
                             isem[2 * par])
            pltpu.async_copy(col_hbm.at[pl.ds(g0 + g * IC, IC), :], colc,
                             isem[2 * par + 1])

        def idx_wait(g, par):
            rowc, colc = idxbufs[par]
            pltpu.make_async_copy(row_hbm.at[pl.ds(g0 + g * IC, IC), :],
                                  rowc, isem[2 * par]).wait()
            pltpu.make_async_copy(col_hbm.at[pl.ds(g0 + g * IC, IC), :],
                                  colc, isem[2 * par + 1]).wait()

        def gather_issue(rowc, lj, b):
            pltpu.async_copy(z_sh.at[rowc.at[lj]], gb[b], gsem[b])

        def gather_wait(rowc, lj, b):
            pltpu.make_async_copy(z_sh.at[rowc.at[lj]], gb[b],
                                  gsem[b]).wait()

        def scatter_issue(colc, lj, b):
            pltpu.async_copy(gb[b], u_sh.at[colc.at[lj]], ssem[b], add=True)

        def scatter_wait(colc, lj, b):
            pltpu.make_async_copy(gb[b], u_sh.at[colc.at[lj]],
                                  ssem[b]).wait()

        def process_group(g, par, first):
            rowc, colc = idxbufs[par]
            idx_wait(g, par)
            def _w0():
                scatter_wait(colc, 0, 0)
            if first:
                pl.when(g >= 1)(_w0)
            else:
                _w0()
            gather_issue(rowc, 0, 0)
            for lj in range(IC):
                b = lj % NBUF
                gather_wait(rowc, lj, b)
                scatter_issue(colc, lj, b)
                if lj < IC - 1:
                    b2 = (lj + 1) % NBUF
                    if lj >= 1:
                        scatter_wait(colc, lj - 1, b2)
                    else:
                        def _w1():
                            scatter_wait(colc, 0, b2)
                        if first:
                            pl.when(g >= 1)(_w1)
                        else:
                            _w1()
                    gather_issue(rowc, lj + 1, b2)

        idx_fetch(0, 0)
        idx_fetch(1, 1)

        def hop(_, carry):
            def pair(p, carry2):
                process_group(2 * p, 0, True)
                @pl.when(p < npair - 1)
                def _():
                    idx_fetch(2 * p + 2, 0)
                process_group(2 * p + 1, 1, False)
                @pl.when(p < npair - 1)
                def _():
                    idx_fetch(2 * p + 3, 1)
                return carry2
            lax.fori_loop(0, npair, pair, 0, unroll=False)

            for lj in range(IC - NBUF, IC):
                scatter_wait(colcb, lj, lj % NBUF)
            idx_fetch(0, 0)
            idx_fetch(1, 1)

            plsc.subcore_barrier()

            def scale_chunk(c, carry2):
                rr = r0 + c * RB
                du = pltpu.async_copy(u_sh.at[pl.ds(rr, RB), :], ubuf,
                                      msem[0])
                dz = pltpu.async_copy(z_sh.at[pl.ds(rr, RB), :], zbuf,
                                      msem[1])
                per_core(
                    lambda: pltpu.async_copy(h0_hbm.at[pl.ds(rr, RB), :],
                                             szbuf, msem[2]),
                    lambda: pltpu.async_copy(h1_hbm.at[pl.ds(rr, RB), :],
                                             szbuf, msem[2]))
                du.wait()
                dz.wait()
                pltpu.make_async_copy(h0_hbm.at[pl.ds(rr, RB), :], szbuf,
                                      msem[2]).wait()
                def scale_row(i, carry3):
                    s = d2b[pl.ds(c * RB + i, 16)][0]
                    def scale_f(f, carry4):
                        sl = pl.ds(f * 16, 16)
                        zv = (ubuf[i, sl] + zbuf[i, sl]) * s
                        zbuf[i, sl] = zv
                        szbuf[i, sl] = szbuf[i, sl] + zv
                        return carry4
                    return lax.fori_loop(0, fpc // 16, scale_f, carry3,
                                         unroll=False)
                lax.fori_loop(0, RB, scale_row, 0, unroll=False)
                du2 = pltpu.async_copy(zerob, u_sh.at[pl.ds(rr, RB), :],
                                       msem[0])
                dz2 = pltpu.async_copy(zbuf, z_sh.at[pl.ds(rr, RB), :],
                                       msem[1])
                per_core(
                    lambda: pltpu.async_copy(szbuf,
                                             h0_hbm.at[pl.ds(rr, RB), :],
                                             msem[2]),
                    lambda: pltpu.async_copy(szbuf,
                                             h1_hbm.at[pl.ds(rr, RB), :],
                                             msem[2]))
                du2.wait()
                dz2.wait()
                pltpu.make_async_copy(szbuf, h0_hbm.at[pl.ds(rr, RB), :],
                                      msem[2]).wait()
                return carry2
            lax.fori_loop(0, nreal, scale_chunk, 0, unroll=False)
            plsc.subcore_barrier()
            return carry
        lax.fori_loop(0, K_HOPS, hop, 0, unroll=False)
        idx_wait(0, 0)
        idx_wait(1, 1)

        def out_chunk(c, carry):
            rr = r0 + c * RB
            per_core(
                lambda: pltpu.sync_copy(h0_hbm.at[pl.ds(rr, RB), :], szbuf),
                lambda: pltpu.sync_copy(h1_hbm.at[pl.ds(rr, RB), :], szbuf))
            def out_row(i, carry2):
                s = dsqb[pl.ds(c * RB + i, 16)][0]
                def out_f(f, carry3):
                    sl = pl.ds(f * 16, 16)
                    szbuf[i, sl] = szbuf[i, sl] * s
                    return carry3
                return lax.fori_loop(0, fpc // 16, out_f, carry2,
                                     unroll=False)
            lax.fori_loop(0, RB, out_row, 0, unroll=False)
            per_core(
                lambda: pltpu.sync_copy(szbuf, h0_hbm.at[pl.ds(rr, RB), :]),
                lambda: pltpu.sync_copy(szbuf, h1_hbm.at[pl.ds(rr, RB), :]))
            return carry
        lax.fori_loop(0, nreal, out_chunk, 0, unroll=False)

    return k(x0, x1, row2d, col2d)


def _matmul_tc(h0, h1, w, b2, *, n, d, c):
    bn = 400
    hd = d // 2

    def mm(h0_ref, h1_ref, w_ref, b_ref, o_ref):
        wv = w_ref[...]
        o_ref[...] = (
            jnp.dot(h0_ref[...], wv[:hd], preferred_element_type=jnp.float32)
            + jnp.dot(h1_ref[...], wv[hd:], preferred_element_type=jnp.float32)
            + b_ref[...])

    return pl.pallas_call(
        mm,
        grid=(n // bn,),
        in_specs=[
            pl.BlockSpec((bn, hd), lambda i: (i, 0)),
            pl.BlockSpec((bn, hd), lambda i: (i, 0)),
            pl.BlockSpec((d, c), lambda i: (0, 0)),
            pl.BlockSpec((1, c), lambda i: (0, 0)),
        ],
        out_specs=pl.BlockSpec((bn, c), lambda i: (i, 0)),
        out_shape=jax.ShapeDtypeStruct((n, c), jnp.float32),
    )(h0, h1, w, b2)


def kernel(x, edge_index, W, b):
    n, d = x.shape
    e = edge_index.shape[1]
    c = W.shape[1]

    cpb = -(-e // (NSUB * CH))
    cpb = -(-cpb // (2 * IC)) * (2 * IC)
    epad = cpb * NSUB * CH
    pad = jnp.full((epad - e,), n, dtype=jnp.int32)
    row2d = jnp.concatenate([edge_index[0], pad]).reshape(cpb * NSUB, CH)
    col2d = jnp.concatenate([edge_index[1], pad]).reshape(cpb * NSUB, CH)

    hd = d // 2
    h0, h1 = _ssgc_sc(x[:, :hd], x[:, hd:], row2d, col2d,
                      n=n, d=d, cpb=cpb)
    return _matmul_tc(h0, h1, W, b.reshape(1, c), n=n, d=d, c=c)
